# baseline (speedup 1.0000x reference)
import functools

import jax
import jax.numpy as jnp
from jax import lax
from jax.experimental import pallas as pl
from jax.experimental.pallas import tpu as pltpu
from jax.experimental.pallas import tpu_sc as plsc

N_NODES = 10000
N_EDGES = 320000
D = 128
HD = 64
NUM_CLASSES = 64

NC = 2
NS = 16
CHUNK = 80
EDGES_PER_TILE = N_EDGES // NS
NCHUNK = EDGES_PER_TILE // CHUNK
N_ACC = 10240
ROWS_PER_TILE = N_ACC // NS
CW = 8
NB = 5
LEAD = 3


def _sc_agg_body(with_count, *refs):
    if with_count:
        (x2_hbm, src_hbm, dst_hbm, zf_hbm, zc_hbm, ones_hbm,
         agg_out, cnt_out,
         src_v, dst_v, sidx_v, rows_v, ones_v, acc_sh, cnt_sh, *sems) = refs
    else:
        (x2_hbm, src_hbm, dst_hbm, zf_hbm,
         agg_out,
         src_v, dst_v, sidx_v, rows_v, acc_sh, *sems) = refs
    gsems, ssems, csem = sems[:NB], sems[NB:2 * NB], sems[2 * NB]
    zsem0, zsem1 = sems[2 * NB + 1], sems[2 * NB + 2]

    c = lax.axis_index("c")
    s = lax.axis_index("s")

    row0 = s * ROWS_PER_TILE
    zf_cp = pltpu.make_async_copy(zf_hbm,
                                  acc_sh.at[pl.ds(row0, ROWS_PER_TILE)], zsem0)
    zf_cp.start()
    if with_count:
        zc_cp = pltpu.make_async_copy(zc_hbm,
                                      cnt_sh.at[pl.ds(row0, ROWS_PER_TILE)],
                                      zsem1)
        zc_cp.start()
        pltpu.sync_copy(ones_hbm, ones_v)

    pltpu.sync_copy(src_hbm.at[s], src_v)
    pltpu.sync_copy(dst_hbm.at[s], dst_v)
    zf_cp.wait()
    if with_count:
        zc_cp.wait()
    plsc.subcore_barrier()

    def xform(j, b):
        for k in range(CHUNK // 16):
            sl = pl.ds(k * 16, 16)
            sidx_v[b, sl] = src_v[j, sl] * 2 + c

    def gather(j, b):
        return pltpu.make_async_copy(x2_hbm.at[sidx_v.at[b]],
                                     rows_v.at[b], gsems[b])

    def scatter(j, b):
        return pltpu.make_async_copy(rows_v.at[b], acc_sh.at[dst_v.at[j]],
                                     ssems[b])

    def cnt_scatter(j):
        return pltpu.make_async_copy(ones_v, cnt_sh.at[dst_v.at[j]], csem)

    for j in range(LEAD):
        xform(j, j % NB)
        gather(j, j % NB).start()

    def outer(g, carry):
        for b in range(NB):
            j = NB * g + b
            jg = j + LEAD
            bg = (b + LEAD) % NB

            @pl.when(jg < NCHUNK)
            def _():
                @pl.when(jg >= NB)
                def _():
                    scatter(jg - NB, bg).wait()
                xform(jg, bg)
                gather(jg, bg).start()

            gather(j, b).wait()
            scatter(j, b).start(add=True)
            if with_count:
                @pl.when(lax.rem(j + c, 2) == 0)
                def _():
                    cnt_scatter(j).start(add=True)

                @pl.when((lax.rem(j + c, 2) == 1) & (j >= NB))
                def _():
                    cnt_scatter(j - NB).wait()
        return carry

    lax.fori_loop(0, NCHUNK // NB, outer, 0)
    for j in range(NCHUNK - NB, NCHUNK):
        scatter(j, j % NB).wait()
        if with_count:
            @pl.when(lax.rem(j + c, 2) == 0)
            def _():
                cnt_scatter(j).wait()
    plsc.subcore_barrier()

    pltpu.sync_copy(acc_sh.at[pl.ds(row0, ROWS_PER_TILE)],
                    agg_out.at[pl.ds(row0, ROWS_PER_TILE), pl.ds(c * HD, HD)])
    if with_count:
        pltpu.sync_copy(cnt_sh.at[pl.ds(row0, ROWS_PER_TILE)],
                        cnt_out.at[c, pl.ds(row0, ROWS_PER_TILE)])


def _make_sc_agg(with_count):
    mesh = plsc.VectorSubcoreMesh(core_axis_name="c", subcore_axis_name="s")
    out_type = [jax.ShapeDtypeStruct((N_ACC, D), jnp.float32)]
    scratch = [
        pltpu.VMEM((NCHUNK, CHUNK), jnp.int32),
        pltpu.VMEM((NCHUNK, CHUNK), jnp.int32),
        pltpu.VMEM((NB, CHUNK), jnp.int32),
        pltpu.VMEM((NB, CHUNK, HD), jnp.float32),
    ]
    if with_count:
        out_type.append(jax.ShapeDtypeStruct((NC, N_ACC, CW), jnp.float32))
        scratch.append(pltpu.VMEM((CHUNK, CW), jnp.float32))
    scratch.append(pltpu.VMEM_SHARED((N_ACC, HD), jnp.float32))
    if with_count:
        scratch.append(pltpu.VMEM_SHARED((N_ACC, CW), jnp.float32))
    scratch.extend([pltpu.SemaphoreType.DMA] * (2 * NB + 3))
    return pl.kernel(
        functools.partial(_sc_agg_body, with_count),
        out_type=out_type,
        mesh=mesh,
        scratch_types=scratch,
        compiler_params=pltpu.CompilerParams(use_tc_tiling_on_sc=False),
    )


def _tc_layer1_body(aggp, cntp, x, wl, b, wr, h_out):
    agg = aggp[...]
    cnt = cntp[0, :, 0:1] + cntp[1, :, 0:1]
    inv = 1.0 / jnp.maximum(cnt, 1.0)
    acc = jnp.dot(agg * inv, wl[...], preferred_element_type=jnp.float32)
    acc += jnp.dot(x[...], wr[...], preferred_element_type=jnp.float32)
    h_out[...] = jnp.maximum(acc + b[...], 0.0)


def _tc_layer2_body(aggp, cntp, h, wl, b, wr, w3, b3, out):
    agg = aggp[...]
    cnt = cntp[0, :, 0:1] + cntp[1, :, 0:1]
    inv = 1.0 / jnp.maximum(cnt, 1.0)
    acc = jnp.dot(agg * inv, wl[...], preferred_element_type=jnp.float32)
    acc += jnp.dot(h[...], wr[...], preferred_element_type=jnp.float32)
    h2 = jnp.maximum(acc + b[...], 0.0)
    out[...] = jnp.dot(h2, w3[...], preferred_element_type=jnp.float32) + b3[...]


_BLK = 2000
_NBLK = N_NODES // _BLK


def _tc_layer1(aggp, cntp, x, wlT, b, wrT):
    return pl.pallas_call(
        _tc_layer1_body,
        grid=(_NBLK,),
        in_specs=[
            pl.BlockSpec((_BLK, D), lambda i: (i, 0)),
            pl.BlockSpec((NC, _BLK, CW), lambda i: (0, i, 0)),
            pl.BlockSpec((_BLK, D), lambda i: (i, 0)),
            pl.BlockSpec((D, D), lambda i: (0, 0)),
            pl.BlockSpec((1, D), lambda i: (0, 0)),
            pl.BlockSpec((D, D), lambda i: (0, 0)),
        ],
        out_specs=pl.BlockSpec((_BLK, D), lambda i: (i, 0)),
        out_shape=jax.ShapeDtypeStruct((N_NODES, D), jnp.float32),
    )(aggp, cntp, x, wlT, b, wrT)


def _tc_layer2(aggp, cntp, h, wlT, b, wrT, w3T, b3):
    return pl.pallas_call(
        _tc_layer2_body,
        grid=(_NBLK,),
        in_specs=[
            pl.BlockSpec((_BLK, D), lambda i: (i, 0)),
            pl.BlockSpec((NC, _BLK, CW), lambda i: (0, i, 0)),
            pl.BlockSpec((_BLK, D), lambda i: (i, 0)),
            pl.BlockSpec((D, D), lambda i: (0, 0)),
            pl.BlockSpec((1, D), lambda i: (0, 0)),
            pl.BlockSpec((D, D), lambda i: (0, 0)),
            pl.BlockSpec((D, NUM_CLASSES), lambda i: (0, 0)),
            pl.BlockSpec((1, NUM_CLASSES), lambda i: (0, 0)),
        ],
        out_specs=pl.BlockSpec((_BLK, NUM_CLASSES), lambda i: (i, 0)),
        out_shape=jax.ShapeDtypeStruct((N_NODES, NUM_CLASSES), jnp.float32),
    )(aggp, cntp, h, wlT, b, wrT, w3T, b3)


_sc_agg_count = _make_sc_agg(True)
_sc_agg = _make_sc_agg(False)


def kernel(x, edge_index, W1l, b1, W1r, W2l, b2, W2r, W3, b3):
    ei = edge_index.astype(jnp.int32)
    srcr = ei[0].reshape(NS, NCHUNK, CHUNK)
    dstr = ei[1].reshape(NS, NCHUNK, CHUNK)
    zf = jnp.zeros((ROWS_PER_TILE, HD), jnp.float32)
    zc = jnp.zeros((ROWS_PER_TILE, CW), jnp.float32)
    ones = jnp.ones((CHUNK, CW), jnp.float32)

    x2 = x.reshape(2 * N_NODES, HD)
    aggp1, cntp = _sc_agg_count(x2, srcr, dstr, zf, zc, ones)
    h = _tc_layer1(aggp1, cntp, x, W1l.T, b1.reshape(1, D), W1r.T)
    h2v = h.reshape(2 * N_NODES, HD)
    (aggp2,) = _sc_agg(h2v, srcr, dstr, zf)
    out = _tc_layer2(aggp2, cntp, h, W2l.T, b2.reshape(1, D), W2r.T,
                     W3.T, b3.reshape(1, NUM_CLASSES))
    return out


# speedup vs baseline: 1.0373x; 1.0373x over previous
import functools

import jax
import jax.numpy as jnp
from jax import lax
from jax.experimental import pallas as pl
from jax.experimental.pallas import tpu as pltpu
from jax.experimental.pallas import tpu_sc as plsc

N_NODES = 10000
N_EDGES = 320000
D = 128
HD = 64
NUM_CLASSES = 64

NC = 2
NS = 16
CHUNK = 80
EDGES_PER_TILE = N_EDGES // NS
NCHUNK = EDGES_PER_TILE // CHUNK
N_ACC = 10240
ROWS_PER_TILE = N_ACC // NS
CW = 8
NB = 5
LEAD = 3


def _sc_agg_body(with_count, *refs):
    if with_count:
        (x2_hbm, src_hbm, dst_hbm, zf_hbm, zc_hbm, ones_hbm,
         agg_out, cnt_out,
         src_v, dst_v, sidx_v, rows_v, ones_v, acc_sh, cnt_sh, *sems) = refs
    else:
        (x2_hbm, src_hbm, dst_hbm, zf_hbm,
         agg_out,
         src_v, dst_v, sidx_v, rows_v, acc_sh, *sems) = refs
    gsems, ssems, csem = sems[:NB], sems[NB:2 * NB], sems[2 * NB]
    zsem0, zsem1 = sems[2 * NB + 1], sems[2 * NB + 2]

    c = lax.axis_index("c")
    s = lax.axis_index("s")

    row0 = s * ROWS_PER_TILE
    zf_cp = pltpu.make_async_copy(zf_hbm.at[pl.ds(row0, ROWS_PER_TILE)],
                                  acc_sh.at[pl.ds(row0, ROWS_PER_TILE)], zsem0)
    zf_cp.start()
    if with_count:
        zc_cp = pltpu.make_async_copy(zc_hbm.at[pl.ds(row0, ROWS_PER_TILE)],
                                      cnt_sh.at[pl.ds(row0, ROWS_PER_TILE)],
                                      zsem1)
        zc_cp.start()
        pltpu.sync_copy(ones_hbm, ones_v)

    pltpu.sync_copy(src_hbm.at[s], src_v)
    pltpu.sync_copy(dst_hbm.at[s], dst_v)
    zf_cp.wait()
    if with_count:
        zc_cp.wait()
    plsc.subcore_barrier()

    def xform(j, b):
        for k in range(CHUNK // 16):
            sl = pl.ds(k * 16, 16)
            sidx_v[b, sl] = src_v[j, sl] * 2 + c

    def gather(j, b):
        return pltpu.make_async_copy(x2_hbm.at[sidx_v.at[b]],
                                     rows_v.at[b], gsems[b])

    def scatter(j, b):
        return pltpu.make_async_copy(rows_v.at[b], acc_sh.at[dst_v.at[j]],
                                     ssems[b])

    def cnt_scatter(j):
        return pltpu.make_async_copy(ones_v, cnt_sh.at[dst_v.at[j]], csem)

    for j in range(LEAD):
        xform(j, j % NB)
        gather(j, j % NB).start()

    def outer(g, carry):
        for b in range(NB):
            j = NB * g + b
            jg = j + LEAD
            bg = (b + LEAD) % NB

            @pl.when(jg < NCHUNK)
            def _():
                @pl.when(jg >= NB)
                def _():
                    scatter(jg - NB, bg).wait()
                xform(jg, bg)
                gather(jg, bg).start()

            gather(j, b).wait()
            scatter(j, b).start(add=True)
            if with_count:
                @pl.when(lax.rem(j + c, 2) == 0)
                def _():
                    cnt_scatter(j).start(add=True)

                @pl.when((lax.rem(j + c, 2) == 1) & (j >= NB))
                def _():
                    cnt_scatter(j - NB).wait()
        return carry

    lax.fori_loop(0, NCHUNK // NB, outer, 0)
    for j in range(NCHUNK - NB, NCHUNK):
        scatter(j, j % NB).wait()
        if with_count:
            @pl.when(lax.rem(j + c, 2) == 0)
            def _():
                cnt_scatter(j).wait()
    plsc.subcore_barrier()

    pltpu.sync_copy(acc_sh.at[pl.ds(row0, ROWS_PER_TILE)],
                    agg_out.at[pl.ds(row0, ROWS_PER_TILE), pl.ds(c * HD, HD)])
    if with_count:
        pltpu.sync_copy(cnt_sh.at[pl.ds(row0, ROWS_PER_TILE)],
                        cnt_out.at[pl.ds(row0, ROWS_PER_TILE), pl.ds(c * CW, CW)])


def _make_sc_agg(with_count):
    mesh = plsc.VectorSubcoreMesh(core_axis_name="c", subcore_axis_name="s")
    out_type = [jax.ShapeDtypeStruct((N_ACC, D), jnp.float32)]
    scratch = [
        pltpu.VMEM((NCHUNK, CHUNK), jnp.int32),
        pltpu.VMEM((NCHUNK, CHUNK), jnp.int32),
        pltpu.VMEM((NB, CHUNK), jnp.int32),
        pltpu.VMEM((NB, CHUNK, HD), jnp.float32),
    ]
    if with_count:
        out_type.append(jax.ShapeDtypeStruct((N_ACC, D), jnp.float32))
        scratch.append(pltpu.VMEM((CHUNK, CW), jnp.float32))
    scratch.append(pltpu.VMEM_SHARED((N_ACC, HD), jnp.float32))
    if with_count:
        scratch.append(pltpu.VMEM_SHARED((N_ACC, CW), jnp.float32))
    scratch.extend([pltpu.SemaphoreType.DMA] * (2 * NB + 3))
    return pl.kernel(
        functools.partial(_sc_agg_body, with_count),
        out_type=out_type,
        mesh=mesh,
        scratch_types=scratch,
        compiler_params=pltpu.CompilerParams(use_tc_tiling_on_sc=False),
    )


def _tc_layer1_body(aggp, cntp, x, wl, b, wr, h_out):
    agg = aggp[...]
    cnt = cntp[:, 0:1] + cntp[:, 8:9]
    inv = 1.0 / jnp.maximum(cnt, 1.0)
    acc = jnp.dot(agg * inv, wl[...], preferred_element_type=jnp.float32)
    acc += jnp.dot(x[...], wr[...], preferred_element_type=jnp.float32)
    h_out[...] = jnp.maximum(acc + b[...], 0.0)


def _tc_layer2_body(aggp, cntp, h, wl, b, wr, w3, b3, out):
    agg = aggp[...]
    cnt = cntp[:, 0:1] + cntp[:, 8:9]
    inv = 1.0 / jnp.maximum(cnt, 1.0)
    acc = jnp.dot(agg * inv, wl[...], preferred_element_type=jnp.float32)
    acc += jnp.dot(h[...], wr[...], preferred_element_type=jnp.float32)
    h2 = jnp.maximum(acc + b[...], 0.0)
    out[...] = jnp.dot(h2, w3[...], preferred_element_type=jnp.float32) + b3[...]


_BLK = 2000
_NBLK = N_NODES // _BLK


def _tc_layer1(aggp, cntp, x, wlT, b, wrT):
    return pl.pallas_call(
        _tc_layer1_body,
        grid=(_NBLK,),
        in_specs=[
            pl.BlockSpec((_BLK, D), lambda i: (i, 0)),
            pl.BlockSpec((_BLK, D), lambda i: (i, 0)),
            pl.BlockSpec((_BLK, D), lambda i: (i, 0)),
            pl.BlockSpec((D, D), lambda i: (0, 0)),
            pl.BlockSpec((1, D), lambda i: (0, 0)),
            pl.BlockSpec((D, D), lambda i: (0, 0)),
        ],
        out_specs=pl.BlockSpec((_BLK, D), lambda i: (i, 0)),
        out_shape=jax.ShapeDtypeStruct((N_NODES, D), jnp.float32),
    )(aggp, cntp, x, wlT, b, wrT)


def _tc_layer2(aggp, cntp, h, wlT, b, wrT, w3T, b3):
    return pl.pallas_call(
        _tc_layer2_body,
        grid=(_NBLK,),
        in_specs=[
            pl.BlockSpec((_BLK, D), lambda i: (i, 0)),
            pl.BlockSpec((_BLK, D), lambda i: (i, 0)),
            pl.BlockSpec((_BLK, D), lambda i: (i, 0)),
            pl.BlockSpec((D, D), lambda i: (0, 0)),
            pl.BlockSpec((1, D), lambda i: (0, 0)),
            pl.BlockSpec((D, D), lambda i: (0, 0)),
            pl.BlockSpec((D, NUM_CLASSES), lambda i: (0, 0)),
            pl.BlockSpec((1, NUM_CLASSES), lambda i: (0, 0)),
        ],
        out_specs=pl.BlockSpec((_BLK, NUM_CLASSES), lambda i: (i, 0)),
        out_shape=jax.ShapeDtypeStruct((N_NODES, NUM_CLASSES), jnp.float32),
    )(aggp, cntp, h, wlT, b, wrT, w3T, b3)


_sc_agg_count = _make_sc_agg(True)
_sc_agg = _make_sc_agg(False)


def kernel(x, edge_index, W1l, b1, W1r, W2l, b2, W2r, W3, b3):
    ei = edge_index.astype(jnp.int32)
    srcr = ei[0].reshape(NS, NCHUNK, CHUNK)
    dstr = ei[1].reshape(NS, NCHUNK, CHUNK)
    zf = jnp.zeros((N_ACC, HD), jnp.float32)
    zc = jnp.zeros((N_ACC, CW), jnp.float32)
    ones = jnp.ones((CHUNK, CW), jnp.float32)

    x2 = x.reshape(2 * N_NODES, HD)
    aggp1, cntp = _sc_agg_count(x2, srcr, dstr, zf, zc, ones)
    h = _tc_layer1(aggp1, cntp, x, W1l.T, b1.reshape(1, D), W1r.T)
    h2v = h.reshape(2 * N_NODES, HD)
    (aggp2,) = _sc_agg(h2v, srcr, dstr, zf)
    out = _tc_layer2(aggp2, cntp, h, W2l.T, b2.reshape(1, D), W2r.T,
                     W3.T, b3.reshape(1, NUM_CLASSES))
    return out


# speedup vs baseline: 1.0493x; 1.0116x over previous
import functools

import jax
import jax.numpy as jnp
from jax import lax
from jax.experimental import pallas as pl
from jax.experimental.pallas import tpu as pltpu
from jax.experimental.pallas import tpu_sc as plsc

N_NODES = 10000
N_EDGES = 320000
D = 128
HD = 64
NUM_CLASSES = 64

NC = 2
NS = 16
CHUNK = 80
EDGES_PER_TILE = N_EDGES // NS
NCHUNK = EDGES_PER_TILE // CHUNK
N_ACC = 10240
ROWS_PER_TILE = N_ACC // NS
CW = 8
NB = 5
LEAD = 4


def _sc_agg_body(with_count, *refs):
    if with_count:
        (x2_hbm, src_hbm, dst_hbm, zf_hbm, zc_hbm, ones_hbm,
         agg_out, cnt_out,
         src_v, dst_v, sidx_v, rows_v, ones_v, acc_sh, cnt_sh, *sems) = refs
    else:
        (x2_hbm, src_hbm, dst_hbm, zf_hbm,
         agg_out,
         src_v, dst_v, sidx_v, rows_v, acc_sh, *sems) = refs
    gsems, ssems, csem = sems[:NB], sems[NB:2 * NB], sems[2 * NB]
    zsem0, zsem1 = sems[2 * NB + 1], sems[2 * NB + 2]

    c = lax.axis_index("c")
    s = lax.axis_index("s")

    row0 = s * ROWS_PER_TILE
    zf_cp = pltpu.make_async_copy(zf_hbm.at[pl.ds(row0, ROWS_PER_TILE)],
                                  acc_sh.at[pl.ds(row0, ROWS_PER_TILE)], zsem0)
    zf_cp.start()
    if with_count:
        zc_cp = pltpu.make_async_copy(zc_hbm.at[pl.ds(row0, ROWS_PER_TILE)],
                                      cnt_sh.at[pl.ds(row0, ROWS_PER_TILE)],
                                      zsem1)
        zc_cp.start()
        pltpu.sync_copy(ones_hbm, ones_v)

    pltpu.sync_copy(src_hbm.at[s], src_v)
    pltpu.sync_copy(dst_hbm.at[s], dst_v)
    zf_cp.wait()
    if with_count:
        zc_cp.wait()
    plsc.subcore_barrier()

    def xform(j, b):
        for k in range(CHUNK // 16):
            sl = pl.ds(k * 16, 16)
            sidx_v[b, sl] = src_v[j, sl] * 2 + c

    def gather(j, b):
        return pltpu.make_async_copy(x2_hbm.at[sidx_v.at[b]],
                                     rows_v.at[b], gsems[b])

    def scatter(j, b):
        return pltpu.make_async_copy(rows_v.at[b], acc_sh.at[dst_v.at[j]],
                                     ssems[b])

    def cnt_scatter(j):
        return pltpu.make_async_copy(ones_v, cnt_sh.at[dst_v.at[j]], csem)

    for j in range(LEAD):
        xform(j, j % NB)
        gather(j, j % NB).start()

    def outer(g, carry):
        for b in range(NB):
            j = NB * g + b
            jg = j + LEAD
            bg = (b + LEAD) % NB

            @pl.when(jg < NCHUNK)
            def _():
                @pl.when(jg >= NB)
                def _():
                    scatter(jg - NB, bg).wait()
                xform(jg, bg)
                gather(jg, bg).start()

            gather(j, b).wait()
            scatter(j, b).start(add=True)
            if with_count:
                @pl.when(lax.rem(j + c, 2) == 0)
                def _():
                    cnt_scatter(j).start(add=True)

                @pl.when((lax.rem(j + c, 2) == 1) & (j >= NB))
                def _():
                    cnt_scatter(j - NB).wait()
        return carry

    lax.fori_loop(0, NCHUNK // NB, outer, 0)
    for j in range(NCHUNK - NB, NCHUNK):
        scatter(j, j % NB).wait()
        if with_count:
            @pl.when(lax.rem(j + c, 2) == 0)
            def _():
                cnt_scatter(j).wait()
    plsc.subcore_barrier()

    pltpu.sync_copy(acc_sh.at[pl.ds(row0, ROWS_PER_TILE)],
                    agg_out.at[pl.ds(row0, ROWS_PER_TILE), pl.ds(c * HD, HD)])
    if with_count:
        pltpu.sync_copy(cnt_sh.at[pl.ds(row0, ROWS_PER_TILE)],
                        cnt_out.at[pl.ds(row0, ROWS_PER_TILE), pl.ds(c * CW, CW)])


def _make_sc_agg(with_count):
    mesh = plsc.VectorSubcoreMesh(core_axis_name="c", subcore_axis_name="s")
    out_type = [jax.ShapeDtypeStruct((N_ACC, D), jnp.float32)]
    scratch = [
        pltpu.VMEM((NCHUNK, CHUNK), jnp.int32),
        pltpu.VMEM((NCHUNK, CHUNK), jnp.int32),
        pltpu.VMEM((NB, CHUNK), jnp.int32),
        pltpu.VMEM((NB, CHUNK, HD), jnp.float32),
    ]
    if with_count:
        out_type.append(jax.ShapeDtypeStruct((N_ACC, D), jnp.float32))
        scratch.append(pltpu.VMEM((CHUNK, CW), jnp.float32))
    scratch.append(pltpu.VMEM_SHARED((N_ACC, HD), jnp.float32))
    if with_count:
        scratch.append(pltpu.VMEM_SHARED((N_ACC, CW), jnp.float32))
    scratch.extend([pltpu.SemaphoreType.DMA] * (2 * NB + 3))
    return pl.kernel(
        functools.partial(_sc_agg_body, with_count),
        out_type=out_type,
        mesh=mesh,
        scratch_types=scratch,
        compiler_params=pltpu.CompilerParams(use_tc_tiling_on_sc=False),
    )


def _tc_layer1_body(aggp, cntp, x, wl, b, wr, h_out):
    agg = aggp[...]
    cnt = cntp[:, 0:1] + cntp[:, 8:9]
    inv = 1.0 / jnp.maximum(cnt, 1.0)
    acc = jnp.dot(agg * inv, wl[...], preferred_element_type=jnp.float32)
    acc += jnp.dot(x[...], wr[...], preferred_element_type=jnp.float32)
    h_out[...] = jnp.maximum(acc + b[...], 0.0)


def _tc_layer2_body(aggp, cntp, h, wl, b, wr, w3, b3, out):
    agg = aggp[...]
    cnt = cntp[:, 0:1] + cntp[:, 8:9]
    inv = 1.0 / jnp.maximum(cnt, 1.0)
    acc = jnp.dot(agg * inv, wl[...], preferred_element_type=jnp.float32)
    acc += jnp.dot(h[...], wr[...], preferred_element_type=jnp.float32)
    h2 = jnp.maximum(acc + b[...], 0.0)
    out[...] = jnp.dot(h2, w3[...], preferred_element_type=jnp.float32) + b3[...]


_BLK = 2000
_NBLK = N_NODES // _BLK


def _tc_layer1(aggp, cntp, x, wlT, b, wrT):
    return pl.pallas_call(
        _tc_layer1_body,
        grid=(_NBLK,),
        in_specs=[
            pl.BlockSpec((_BLK, D), lambda i: (i, 0)),
            pl.BlockSpec((_BLK, D), lambda i: (i, 0)),
            pl.BlockSpec((_BLK, D), lambda i: (i, 0)),
            pl.BlockSpec((D, D), lambda i: (0, 0)),
            pl.BlockSpec((1, D), lambda i: (0, 0)),
            pl.BlockSpec((D, D), lambda i: (0, 0)),
        ],
        out_specs=pl.BlockSpec((_BLK, D), lambda i: (i, 0)),
        out_shape=jax.ShapeDtypeStruct((N_NODES, D), jnp.float32),
    )(aggp, cntp, x, wlT, b, wrT)


def _tc_layer2(aggp, cntp, h, wlT, b, wrT, w3T, b3):
    return pl.pallas_call(
        _tc_layer2_body,
        grid=(_NBLK,),
        in_specs=[
            pl.BlockSpec((_BLK, D), lambda i: (i, 0)),
            pl.BlockSpec((_BLK, D), lambda i: (i, 0)),
            pl.BlockSpec((_BLK, D), lambda i: (i, 0)),
            pl.BlockSpec((D, D), lambda i: (0, 0)),
            pl.BlockSpec((1, D), lambda i: (0, 0)),
            pl.BlockSpec((D, D), lambda i: (0, 0)),
            pl.BlockSpec((D, NUM_CLASSES), lambda i: (0, 0)),
            pl.BlockSpec((1, NUM_CLASSES), lambda i: (0, 0)),
        ],
        out_specs=pl.BlockSpec((_BLK, NUM_CLASSES), lambda i: (i, 0)),
        out_shape=jax.ShapeDtypeStruct((N_NODES, NUM_CLASSES), jnp.float32),
    )(aggp, cntp, h, wlT, b, wrT, w3T, b3)


_sc_agg_count = _make_sc_agg(True)
_sc_agg = _make_sc_agg(False)


def kernel(x, edge_index, W1l, b1, W1r, W2l, b2, W2r, W3, b3):
    ei = edge_index.astype(jnp.int32)
    srcr = ei[0].reshape(NS, NCHUNK, CHUNK)
    dstr = ei[1].reshape(NS, NCHUNK, CHUNK)
    zf = jnp.zeros((N_ACC, HD), jnp.float32)
    zc = jnp.zeros((N_ACC, CW), jnp.float32)
    ones = jnp.ones((CHUNK, CW), jnp.float32)

    x2 = x.reshape(2 * N_NODES, HD)
    aggp1, cntp = _sc_agg_count(x2, srcr, dstr, zf, zc, ones)
    h = _tc_layer1(aggp1, cntp, x, W1l.T, b1.reshape(1, D), W1r.T)
    h2v = h.reshape(2 * N_NODES, HD)
    (aggp2,) = _sc_agg(h2v, srcr, dstr, zf)
    out = _tc_layer2(aggp2, cntp, h, W2l.T, b2.reshape(1, D), W2r.T,
                     W3.T, b3.reshape(1, NUM_CLASSES))
    return out
